# baseline (speedup 1.0000x reference)
import jax
import jax.numpy as jnp
from jax import lax
from jax.experimental import pallas as pl
from jax.experimental.pallas import tpu as pltpu
from jax.experimental.pallas import tpu_sc as plsc

_CLS = 100000
_DIM = 32
_BATCH = 16384
_NS = 16
_PER = _BATCH // _NS
_CHUNK = 128
_NCHUNK = _PER // _CHUNK
_CNT_PAD = 100096
_ZCHUNK = _CNT_PAD // _NS


def _body(ys_ref, xs_ref, center_ref, zeros_ref, ones_ref, out_ref,
          idx_v, xs_v, c_v, cnt_v, ones_v, acc_v, z_v, cnt_sh, sem, sem_z):
    s = lax.axis_index("s")
    base = s * _PER

    zcopy = pltpu.async_copy(zeros_ref.at[pl.ds(s * _ZCHUNK, _ZCHUNK)],
                             z_v, sem_z)
    pltpu.sync_copy(ys_ref.at[pl.ds(s * _NCHUNK, _NCHUNK)], idx_v)
    pltpu.sync_copy(ones_ref, ones_v)

    data_copies = [pltpu.async_copy(xs_ref.at[pl.ds(base, _PER)], xs_v, sem)]
    for g in range(_NCHUNK):
        data_copies.append(pltpu.async_copy(
            center_ref.at[idx_v.at[g]],
            c_v.at[pl.ds(g * _CHUNK, _CHUNK)], sem))

    zcopy.wait()
    pltpu.sync_copy(z_v, cnt_sh.at[pl.ds(s * _ZCHUNK, _ZCHUNK)])

    plsc.subcore_barrier()
    for g in range(_NCHUNK):
        pltpu.sync_copy(ones_v, cnt_sh.at[idx_v.at[g]], add=True)
    plsc.subcore_barrier()
    for g in range(_NCHUNK):
        pltpu.sync_copy(cnt_sh.at[idx_v.at[g]],
                        cnt_v.at[pl.ds(g * _CHUNK, _CHUNK)])
    for c in data_copies:
        c.wait()

    lanes = lax.iota(jnp.int32, 16)

    def group(g, acc):
        rows = lanes + g * 16
        cnt16 = plsc.load_gather(cnt_v, [rows])
        w16 = 0.5 / (cnt16 + 1.0)
        sq = jnp.zeros((16,), jnp.float32)
        for d in range(_DIM):
            col = jnp.full((16,), d, jnp.int32)
            t = (plsc.load_gather(xs_v, [rows, col])
                 - plsc.load_gather(c_v, [rows, col]))
            sq = sq + t * t
        return acc + sq * w16

    acc = lax.fori_loop(0, _PER // 16, group, jnp.zeros((16,), jnp.float32))
    acc_v[...] = acc
    pltpu.sync_copy(acc_v, out_ref.at[s])


def kernel(xs, ys, center):
    ys2d = ys.astype(jnp.int32).reshape(_NS * _NCHUNK, _CHUNK)
    zeros = jnp.zeros((_CNT_PAD,), jnp.float32)
    ones = jnp.ones((_CHUNK,), jnp.float32)
    mesh = plsc.VectorSubcoreMesh(
        core_axis_name="c", subcore_axis_name="s", num_cores=1)
    out = pl.kernel(
        _body,
        out_type=jax.ShapeDtypeStruct((_NS, 16), jnp.float32),
        mesh=mesh,
        compiler_params=pltpu.CompilerParams(
            needs_layout_passes=False, use_tc_tiling_on_sc=False),
        scratch_types=[
            pltpu.VMEM((_NCHUNK, _CHUNK), jnp.int32),
            pltpu.VMEM((_PER, _DIM), jnp.float32),
            pltpu.VMEM((_PER, _DIM), jnp.float32),
            pltpu.VMEM((_PER,), jnp.float32),
            pltpu.VMEM((_CHUNK,), jnp.float32),
            pltpu.VMEM((16,), jnp.float32),
            pltpu.VMEM((_ZCHUNK,), jnp.float32),
            pltpu.VMEM_SHARED((_CNT_PAD,), jnp.float32),
            pltpu.SemaphoreType.DMA,
            pltpu.SemaphoreType.DMA,
        ],
    )(ys2d, xs, center, zeros, ones)
    return jnp.sum(out)


# speedup vs baseline: 1.1603x; 1.1603x over previous
import jax
import jax.numpy as jnp
from jax import lax
from jax.experimental import pallas as pl
from jax.experimental.pallas import tpu as pltpu
from jax.experimental.pallas import tpu_sc as plsc

_CLS = 100000
_DIM = 32
_BATCH = 16384
_NC = 2
_NS = 16
_NW = _NC * _NS
_PER = _BATCH // _NW
_HIST = _BATCH // _NS
_CHUNK = 128
_NHC = _HIST // _CHUNK
_NGC = _PER // _CHUNK
_CNT_PAD = 100096
_ZCHUNK = _CNT_PAD // _NS


def _body(ys_ref, xs_ref, center_ref, zeros_ref, ones_ref, out_ref,
          idx_v, xs_v, c_v, cnt_v, ones_v, acc_v, z_v, cnt_sh,
          sem, sem_z, sem_h):
    c = lax.axis_index("c")
    s = lax.axis_index("s")
    wid = s * _NC + c
    base = wid * _PER
    hbase = s * _HIST
    coff = c * _NGC

    zcopy = pltpu.async_copy(zeros_ref.at[pl.ds(s * _ZCHUNK, _ZCHUNK)],
                             z_v, sem_z)
    pltpu.sync_copy(ys_ref.at[pl.ds(s * _NHC, _NHC)], idx_v)
    pltpu.sync_copy(ones_ref, ones_v)

    data_copies = [pltpu.async_copy(xs_ref.at[pl.ds(base, _PER)], xs_v, sem)]
    for g in range(_NGC):
        data_copies.append(pltpu.async_copy(
            center_ref.at[idx_v.at[coff + g]],
            c_v.at[pl.ds(g * _CHUNK, _CHUNK)], sem))

    zcopy.wait()
    pltpu.sync_copy(z_v, cnt_sh.at[pl.ds(s * _ZCHUNK, _ZCHUNK)])

    plsc.subcore_barrier()
    hist_copies = [
        pltpu.async_copy(ones_v, cnt_sh.at[idx_v.at[g]], sem_h, add=True)
        for g in range(_NHC)
    ]
    for h in hist_copies:
        h.wait()
    plsc.subcore_barrier()
    cnt_copies = [
        pltpu.async_copy(cnt_sh.at[idx_v.at[coff + g]],
                         cnt_v.at[pl.ds(g * _CHUNK, _CHUNK)], sem_h)
        for g in range(_NGC)
    ]
    for cc in cnt_copies:
        cc.wait()
    for dc in data_copies:
        dc.wait()

    lanes = lax.iota(jnp.int32, 16)

    def group(g, acc):
        rows = lanes + g * 16
        cnt16 = plsc.load_gather(cnt_v, [rows])
        w16 = 0.5 / (cnt16 + 1.0)
        sq = jnp.zeros((16,), jnp.float32)
        for d in range(_DIM):
            col = jnp.full((16,), d, jnp.int32)
            t = (plsc.load_gather(xs_v, [rows, col])
                 - plsc.load_gather(c_v, [rows, col]))
            sq = sq + t * t
        return acc + sq * w16

    acc = lax.fori_loop(0, _PER // 16, group, jnp.zeros((16,), jnp.float32))
    acc_v[...] = acc
    pltpu.sync_copy(acc_v, out_ref.at[wid])


def kernel(xs, ys, center):
    ys2d = ys.astype(jnp.int32).reshape(_NS * _NHC, _CHUNK)
    zeros = jnp.zeros((_CNT_PAD,), jnp.float32)
    ones = jnp.ones((_CHUNK,), jnp.float32)
    mesh = plsc.VectorSubcoreMesh(
        core_axis_name="c", subcore_axis_name="s", num_cores=_NC)
    out = pl.kernel(
        _body,
        out_type=jax.ShapeDtypeStruct((_NW, 16), jnp.float32),
        mesh=mesh,
        compiler_params=pltpu.CompilerParams(
            needs_layout_passes=False, use_tc_tiling_on_sc=False),
        scratch_types=[
            pltpu.VMEM((_NHC, _CHUNK), jnp.int32),
            pltpu.VMEM((_PER, _DIM), jnp.float32),
            pltpu.VMEM((_PER, _DIM), jnp.float32),
            pltpu.VMEM((_PER,), jnp.float32),
            pltpu.VMEM((_CHUNK,), jnp.float32),
            pltpu.VMEM((16,), jnp.float32),
            pltpu.VMEM((_ZCHUNK,), jnp.float32),
            pltpu.VMEM_SHARED((_CNT_PAD,), jnp.float32),
            pltpu.SemaphoreType.DMA,
            pltpu.SemaphoreType.DMA,
            pltpu.SemaphoreType.DMA,
        ],
    )(ys2d, xs, center, zeros, ones)
    return jnp.sum(out)


# speedup vs baseline: 1.1652x; 1.0042x over previous
import jax
import jax.numpy as jnp
from jax import lax
from jax.experimental import pallas as pl
from jax.experimental.pallas import tpu as pltpu
from jax.experimental.pallas import tpu_sc as plsc

_CLS = 100000
_DIM = 32
_BATCH = 16384
_NC = 2
_NS = 16
_NW = _NC * _NS
_PER = _BATCH // _NW
_HIST = _BATCH // _NS
_CHUNK = 128
_NHC = _HIST // _CHUNK
_NGC = _PER // _CHUNK
_CNT_PAD = 100096
_ZCHUNK = _CNT_PAD // _NS


def _body(ys_ref, xs_ref, center_ref, out_ref,
          idx_v, idx4_v, xs_v, c_v, cnt_v, ones_v, z_v, acc_v, cnt_sh,
          sem, sem_i, sem_h):
    c = lax.axis_index("c")
    s = lax.axis_index("s")
    wid = s * _NC + c
    lanes = lax.iota(jnp.int32, 16)
    zero16 = jnp.zeros((16,), jnp.float32)

    icopy = pltpu.async_copy(ys_ref.at[pl.ds(s * _NHC, _NHC)], idx_v, sem_i)

    for k in range(_CHUNK // 16):
        ones_v[pl.ds(k * 16, 16)] = zero16 + 1.0

    def zstore(i, carry):
        z_v[pl.ds(i * 16, 16)] = zero16
        return carry

    lax.fori_loop(0, _ZCHUNK // 16, zstore, 0)
    icopy.wait()

    for g in range(_NGC):
        for k in range(_CHUNK // 16):
            y16 = idx_v[c * _NGC + g, pl.ds(k * 16, 16)]
            idx4_v[g, pl.ds(k * 16, 16)] = y16 >> 2

    data_copies = [pltpu.async_copy(
        xs_ref.at[pl.ds(wid * (_PER // 4), _PER // 4)], xs_v, sem)]
    for g in range(_NGC):
        data_copies.append(pltpu.async_copy(
            center_ref.at[idx4_v.at[g]],
            c_v.at[pl.ds(g * _CHUNK, _CHUNK)], sem))

    pltpu.sync_copy(z_v, cnt_sh.at[pl.ds(s * _ZCHUNK, _ZCHUNK)])

    plsc.subcore_barrier()
    hist_copies = [
        pltpu.async_copy(ones_v, cnt_sh.at[idx_v.at[g]], sem_h, add=True)
        for g in range(_NHC)
    ]
    for h in hist_copies:
        h.wait()
    plsc.subcore_barrier()
    cnt_copies = [
        pltpu.async_copy(cnt_sh.at[idx_v.at[c * _NGC + g]],
                         cnt_v.at[pl.ds(g * _CHUNK, _CHUNK)], sem_h)
        for g in range(_NGC)
    ]
    for cc in cnt_copies:
        cc.wait()
    for dc in data_copies:
        dc.wait()

    xcol = [((lanes & 3) << 5) + d for d in range(_DIM)]

    def group(g, acc):
        rows = lanes + g * 16
        rowsx = rows >> 2
        y16 = idx_v[c * _NGC + (g >> 3), pl.ds((g & 7) * 16, 16)]
        ccol0 = (y16 & 3) << 5
        cnt16 = plsc.load_gather(cnt_v, [rows])
        w16 = 0.5 / (cnt16 + 1.0)
        sq = zero16
        for d in range(_DIM):
            t = (plsc.load_gather(xs_v, [rowsx, xcol[d]])
                 - plsc.load_gather(c_v, [rows, ccol0 + d]))
            sq = sq + t * t
        return acc + sq * w16

    acc = lax.fori_loop(0, _PER // 16, group, zero16)
    acc_v[...] = acc
    pltpu.sync_copy(acc_v, out_ref.at[pl.ds(wid * 16, 16)])


def kernel(xs, ys, center):
    ys2d = ys.astype(jnp.int32).reshape(_NS * _NHC, _CHUNK)
    xs128 = xs.reshape(_BATCH // 4, 128)
    c128 = center.reshape(_CLS // 4, 128)
    mesh = plsc.VectorSubcoreMesh(
        core_axis_name="c", subcore_axis_name="s", num_cores=_NC)
    out = pl.kernel(
        _body,
        out_type=jax.ShapeDtypeStruct((_NW * 16,), jnp.float32),
        mesh=mesh,
        compiler_params=pltpu.CompilerParams(
            needs_layout_passes=False, use_tc_tiling_on_sc=True),
        scratch_types=[
            pltpu.VMEM((_NHC, _CHUNK), jnp.int32),
            pltpu.VMEM((_NGC, _CHUNK), jnp.int32),
            pltpu.VMEM((_PER // 4, 128), jnp.float32),
            pltpu.VMEM((_PER, 128), jnp.float32),
            pltpu.VMEM((_PER,), jnp.float32),
            pltpu.VMEM((_CHUNK,), jnp.float32),
            pltpu.VMEM((_ZCHUNK,), jnp.float32),
            pltpu.VMEM((16,), jnp.float32),
            pltpu.VMEM_SHARED((_CNT_PAD,), jnp.float32),
            pltpu.SemaphoreType.DMA,
            pltpu.SemaphoreType.DMA,
            pltpu.SemaphoreType.DMA,
        ],
    )(ys2d, xs128, c128)
    return jnp.sum(out)
